# baseline (speedup 1.0000x reference)
import jax
import jax.numpy as jnp
from jax import lax
from jax.experimental import pallas as pl
from jax.experimental.pallas import tpu as pltpu

_ROWS = 16384
_HIDDEN = 2048
_EXPERTS = 16
_BLOCK = 512


def _router_block(x_ref, w_ref, tw_ref, ti_ref, logits_ref):
    x = x_ref[...]
    w = w_ref[...]
    logits_t = jax.lax.dot_general(
        w, x, (((1,), (1,)), ((), ())), preferred_element_type=jnp.float32
    )
    logits_ref[...] = logits_t

    e_iota = jax.lax.broadcasted_iota(jnp.int32, logits_t.shape, 0)
    m1 = jnp.max(logits_t, axis=0, keepdims=True)
    i1 = jnp.min(jnp.where(logits_t == m1, e_iota, _EXPERTS), axis=0, keepdims=True)
    masked = jnp.where(e_iota == i1, -jnp.inf, logits_t)
    m2 = jnp.max(masked, axis=0, keepdims=True)
    i2 = jnp.min(jnp.where(masked == m2, e_iota, _EXPERTS), axis=0, keepdims=True)

    e2 = jnp.exp(m2 - m1)
    w1 = 1.0 / (1.0 + e2)
    w2 = e2 / (1.0 + e2)

    k_iota = jax.lax.broadcasted_iota(jnp.int32, (2, logits_t.shape[1]), 0)
    tw_ref[...] = jnp.where(k_iota == 0, w1, w2)
    ti_ref[...] = jnp.where(k_iota == 0, i1, i2)


def kernel(hidden_states, W_gate):
    grid = (_ROWS // _BLOCK,)
    tw_t, ti_t, logits_t = pl.pallas_call(
        _router_block,
        grid=grid,
        in_specs=[
            pl.BlockSpec((_BLOCK, _HIDDEN), lambda i: (i, 0)),
            pl.BlockSpec((_EXPERTS, _HIDDEN), lambda i: (0, 0)),
        ],
        out_specs=[
            pl.BlockSpec((2, _BLOCK), lambda i: (0, i)),
            pl.BlockSpec((2, _BLOCK), lambda i: (0, i)),
            pl.BlockSpec((_EXPERTS, _BLOCK), lambda i: (0, i)),
        ],
        out_shape=[
            jax.ShapeDtypeStruct((2, _ROWS), jnp.float32),
            jax.ShapeDtypeStruct((2, _ROWS), jnp.int32),
            jax.ShapeDtypeStruct((_EXPERTS, _ROWS), jnp.float32),
        ],
        compiler_params=pltpu.CompilerParams(
            dimension_semantics=("parallel",),
        ),
    )(hidden_states, W_gate)
    return (tw_t.T, ti_t.T, logits_t.T)


# speedup vs baseline: 1.1865x; 1.1865x over previous
import jax
import jax.numpy as jnp
from jax import lax
from jax.experimental import pallas as pl
from jax.experimental.pallas import tpu as pltpu

_ROWS = 16384
_HIDDEN = 2048
_EXPERTS = 16
_BLOCK = 1024


def _router_block(x_ref, w_ref, tw_ref, ti_ref, logits_ref):
    x = x_ref[...]
    w = w_ref[...]
    logits_t = jax.lax.dot_general(
        w, x, (((1,), (1,)), ((), ())), preferred_element_type=jnp.float32
    )
    logits_ref[...] = logits_t

    e_iota = jax.lax.broadcasted_iota(jnp.int32, logits_t.shape, 0)
    m1 = jnp.max(logits_t, axis=0, keepdims=True)
    i1 = jnp.min(jnp.where(logits_t == m1, e_iota, _EXPERTS), axis=0, keepdims=True)
    masked = jnp.where(e_iota == i1, -jnp.inf, logits_t)
    m2 = jnp.max(masked, axis=0, keepdims=True)
    i2 = jnp.min(jnp.where(masked == m2, e_iota, _EXPERTS), axis=0, keepdims=True)

    e2 = jnp.exp(m2 - m1)
    w1 = 1.0 / (1.0 + e2)
    w2 = e2 / (1.0 + e2)

    k_iota = jax.lax.broadcasted_iota(jnp.int32, (2, logits_t.shape[1]), 0)
    tw_ref[...] = jnp.where(k_iota == 0, w1, w2)
    ti_ref[...] = jnp.where(k_iota == 0, i1, i2)


def kernel(hidden_states, W_gate):
    grid = (_ROWS // _BLOCK,)
    tw_t, ti_t, logits_t = pl.pallas_call(
        _router_block,
        grid=grid,
        in_specs=[
            pl.BlockSpec((_BLOCK, _HIDDEN), lambda i: (i, 0)),
            pl.BlockSpec((_EXPERTS, _HIDDEN), lambda i: (0, 0)),
        ],
        out_specs=[
            pl.BlockSpec((2, _BLOCK), lambda i: (0, i)),
            pl.BlockSpec((2, _BLOCK), lambda i: (0, i)),
            pl.BlockSpec((_EXPERTS, _BLOCK), lambda i: (0, i)),
        ],
        out_shape=[
            jax.ShapeDtypeStruct((2, _ROWS), jnp.float32),
            jax.ShapeDtypeStruct((2, _ROWS), jnp.int32),
            jax.ShapeDtypeStruct((_EXPERTS, _ROWS), jnp.float32),
        ],
        compiler_params=pltpu.CompilerParams(
            dimension_semantics=("parallel",),
        ),
    )(hidden_states, W_gate)
    return (tw_t.T, ti_t.T, logits_t.T)


# speedup vs baseline: 1.2118x; 1.0213x over previous
import jax
import jax.numpy as jnp
from jax.experimental import pallas as pl
from jax.experimental.pallas import tpu as pltpu

_ROWS = 16384
_HIDDEN = 2048
_EXPERTS = 16
_BLOCK = 1024


def _router_block(x_ref, w_ref, tw_ref, ti_ref, logits_ref):
    x = x_ref[...]
    w = w_ref[...]
    logits_t = jax.lax.dot_general(
        w, x, (((1,), (1,)), ((), ())), preferred_element_type=jnp.float32
    )
    logits_ref[...] = logits_t

    e_iota = jax.lax.broadcasted_iota(jnp.int32, logits_t.shape, 0)
    m1 = jnp.max(logits_t, axis=0, keepdims=True)
    i1 = jnp.min(jnp.where(logits_t == m1, e_iota, _EXPERTS), axis=0, keepdims=True)
    masked = jnp.where(e_iota == i1, -jnp.inf, logits_t)
    m2 = jnp.max(masked, axis=0, keepdims=True)
    i2 = jnp.min(jnp.where(masked == m2, e_iota, _EXPERTS), axis=0, keepdims=True)

    e2 = jnp.exp(m2 - m1)
    w1 = 1.0 / (1.0 + e2)
    w2 = e2 / (1.0 + e2)

    k_iota = jax.lax.broadcasted_iota(jnp.int32, (2, logits_t.shape[1]), 0)
    tw_ref[...] = jnp.where(k_iota == 0, w1, w2)
    ti_ref[...] = jnp.where(k_iota == 0, i1, i2)


def kernel(hidden_states, W_gate):
    grid = (_ROWS // _BLOCK,)
    tw_t, ti_t, logits_t = pl.pallas_call(
        _router_block,
        grid=grid,
        in_specs=[
            pl.BlockSpec((_BLOCK, _HIDDEN), lambda i: (i, 0)),
            pl.BlockSpec((_EXPERTS, _HIDDEN), lambda i: (0, 0)),
        ],
        out_specs=[
            pl.BlockSpec((2, _BLOCK), lambda i: (0, i)),
            pl.BlockSpec((2, _BLOCK), lambda i: (0, i)),
            pl.BlockSpec((_EXPERTS, _BLOCK), lambda i: (0, i)),
        ],
        out_shape=[
            jax.ShapeDtypeStruct((2, _ROWS), jnp.float32),
            jax.ShapeDtypeStruct((2, _ROWS), jnp.int32),
            jax.ShapeDtypeStruct((_EXPERTS, _ROWS), jnp.float32),
        ],
        compiler_params=pltpu.CompilerParams(
            dimension_semantics=("parallel",),
        ),
    )(hidden_states, W_gate)
    return (tw_t.T, ti_t.T, logits_t.T)
